# Initial kernel scaffold; baseline (speedup 1.0000x reference)
import jax
import jax.numpy as jnp
from jax.experimental import pallas as pl


def kernel(input_ids, table):
    raise NotImplementedError("write your pallas kernel here")


# speedup vs baseline: 1.5119x; 1.5119x over previous
import functools

import jax
import jax.numpy as jnp
from jax import lax
from jax.experimental import pallas as pl
from jax.experimental.pallas import tpu as pltpu
from jax.experimental.pallas import tpu_sc as plsc

NUM_BUCKETS = 1000000
EMBED_DIM = 32
ROW = 200
NROWS = 4096
TOTAL = NROWS * ROW
NW = 32
IDS_PER_W = TOTAL // NW
CHUNK = 1600
NCHUNK = IDS_PER_W // CHUNK
NVREG = CHUNK // 16

C_PH = 761000
C_PL = 435761
C_CH = 503000
C_CL = 40503


def _make_sc_call():
    mesh = plsc.VectorSubcoreMesh(core_axis_name="c", subcore_axis_name="s")

    @functools.partial(
        pl.kernel,
        mesh=mesh,
        out_type=jax.ShapeDtypeStruct((TOTAL, EMBED_DIM), jnp.float32),
        scratch_types=[
            pltpu.VMEM((CHUNK + 16,), jnp.int32),
            pltpu.VMEM((CHUNK,), jnp.int32),
            pltpu.VMEM((CHUNK, EMBED_DIM), jnp.float32),
            pltpu.SemaphoreType.DMA,
        ],
        compiler_params=pltpu.CompilerParams(use_tc_tiling_on_sc=False),
    )
    def sc_gather(ids_hbm, table_hbm, out_hbm, ids_v, idx_v, rows_v, sem):
        wid = lax.axis_index("s") * 2 + lax.axis_index("c")
        base = wid * IDS_PER_W

        def chunk_body(ci, carry):
            off = base + ci * jnp.int32(CHUNK)
            pltpu.sync_copy(ids_hbm.at[pl.ds(off, CHUNK)],
                            ids_v.at[pl.ds(8, CHUNK)])

            def hash_body(k, c):
                k16 = k * jnp.int32(16)
                cur = ids_v[pl.ds(jnp.int32(8) + k16, 16)]
                prev = ids_v[pl.ds(jnp.int32(7) + k16, 16)]
                lp = k16 + lax.iota(jnp.int32, 16)
                prev = jnp.where(lp % jnp.int32(ROW) == 0, 0, prev)
                ph = lax.div(prev, jnp.int32(1000))
                plo = prev - ph * jnp.int32(1000)
                ch = lax.div(cur, jnp.int32(1000))
                clo = cur - ch * jnp.int32(1000)
                s = (jnp.int32(C_PH) * ph + jnp.int32(C_PL) * plo
                     + jnp.int32(C_CH) * ch + jnp.int32(C_CL) * clo)
                idx_v[pl.ds(k16, 16)] = s % jnp.int32(NUM_BUCKETS)
                return c

            lax.fori_loop(jnp.int32(0), jnp.int32(NVREG), hash_body,
                          jnp.int32(0))

            copies = []
            for j in range(CHUNK // 128):
                copies.append(pltpu.async_copy(
                    table_hbm.at[idx_v.at[pl.ds(j * 128, 128)]],
                    rows_v.at[pl.ds(j * 128, 128)],
                    sem))
            rem = CHUNK % 128
            if rem:
                j = CHUNK // 128
                copies.append(pltpu.async_copy(
                    table_hbm.at[idx_v.at[pl.ds(j * 128, rem)]],
                    rows_v.at[pl.ds(j * 128, rem)],
                    sem))
            for c in copies:
                c.wait()

            pltpu.sync_copy(rows_v, out_hbm.at[pl.ds(off, CHUNK)])
            return carry

        lax.fori_loop(jnp.int32(0), jnp.int32(NCHUNK), chunk_body,
                      jnp.int32(0))

    return sc_gather


_SC_GATHER = _make_sc_call()


def kernel(input_ids, table):
    ids32 = input_ids.reshape(-1).astype(jnp.int32)
    out = _SC_GATHER(ids32, table)
    return out.reshape(NROWS, ROW, EMBED_DIM)
